# baseline (speedup 1.0000x reference)
import jax
import jax.numpy as jnp
from jax import lax
from jax.experimental import pallas as pl
from jax.experimental.pallas import tpu as pltpu
from jax.experimental.pallas import tpu_sc as plsc

NE = 320000
D = 128
NSEG = 10000
NC = 2
NS = 16
L = 16

SEG_PAD = 10240
RPT = SEG_PAD // NS

IDX_COLS = 128
IDX_ROWS = NE // IDX_COLS
CORE_ROWS = IDX_ROWS // NC
BASE_ROWS = CORE_ROWS // NS
EXTRA = CORE_ROWS - BASE_ROWS * NS
STEPS = BASE_ROWS
CHUNK = IDX_COLS


def _sc_body(msg_hbm, idx_hbm, psum_hbm, pcnt_hbm, acc,
             b0, b1, i0, i1, counts,
             si0, sm0, si1, sm1, sa0, sa1):
    c = lax.axis_index("c")
    s = lax.axis_index("s")
    seg0 = s * RPT
    row_base = c * CORE_ROWS + s * BASE_ROWS

    zero16 = jnp.zeros((L,), jnp.float32)
    ones16 = jnp.full((L,), 1.0, jnp.float32)

    def _zb(i, carry):
        b0[i // (D // L), pl.ds((i % (D // L)) * L, L)] = zero16
        return carry
    lax.fori_loop(0, CHUNK * (D // L), _zb, None)
    def _za(q, carry):
        pltpu.sync_copy(b0, acc.at[pl.ds(seg0 + q * CHUNK, CHUNK)])
        return carry
    lax.fori_loop(0, RPT // CHUNK, _za, None)
    def _zc(i, carry):
        counts[pl.ds(i * L, L)] = zero16
        return carry
    lax.fori_loop(0, SEG_PAD // L, _zc, None)
    plsc.subcore_barrier()

    def _load(row, ib, buf, si, sm):
        pltpu.async_copy(idx_hbm.at[pl.ds(row, 1)], ib, si)
        pltpu.async_copy(msg_hbm.at[pl.ds(row * IDX_COLS, CHUNK)], buf, sm)

    def _wait_load(row, ib, buf, si, sm):
        pltpu.make_async_copy(idx_hbm.at[pl.ds(row, 1)], ib, si).wait()
        pltpu.make_async_copy(msg_hbm.at[pl.ds(row * IDX_COLS, CHUNK)],
                              buf, sm).wait()

    def _count(ib):
        for q in range(IDX_COLS // L):
            iv = ib[0, pl.ds(q * L, L)]
            plsc.addupdate_scatter(counts, [iv], ones16)

    _load(row_base, i0, b0, si0, sm0)
    def _pair(p, carry):
        row_a = row_base + 2 * p
        _load(row_a + 1, i1, b1, si1, sm1)
        _wait_load(row_a, i0, b0, si0, sm0)
        d0 = pltpu.async_copy(b0, acc.at[i0.at[0]], sa0, add=True)
        _count(i0)
        d0.wait()

        @pl.when(p < STEPS // 2 - 1)
        def _():
            _load(row_a + 2, i0, b0, si0, sm0)
        _wait_load(row_a + 1, i1, b1, si1, sm1)
        d1 = pltpu.async_copy(b1, acc.at[i1.at[0]], sa1, add=True)
        _count(i1)
        d1.wait()
        return carry
    lax.fori_loop(0, STEPS // 2, _pair, None)

    @pl.when(s < EXTRA)
    def _extra():
        row = c * CORE_ROWS + NS * BASE_ROWS + s
        pltpu.sync_copy(idx_hbm.at[pl.ds(row, 1)], i0)
        pltpu.sync_copy(msg_hbm.at[pl.ds(row * IDX_COLS, CHUNK)], b0)
        pltpu.sync_copy(b0, acc.at[i0.at[0]], add=True)
        _count(i0)

    pltpu.sync_copy(counts, pcnt_hbm.at[c, s])
    plsc.subcore_barrier()
    pltpu.sync_copy(acc.at[pl.ds(seg0, RPT)],
                    psum_hbm.at[c, pl.ds(seg0, RPT)])


N_BLK = 1024


def _combine_body(p_ref, cnt_ref, o_ref):
    cnt = jnp.sum(cnt_ref[...], axis=0)
    total = p_ref[0] + p_ref[1]
    o_ref[...] = total / jnp.maximum(cnt, 1.0)[:, None]


@jax.jit
def kernel(msg, index, t):
    del t
    idx2d = index.astype(jnp.int32).reshape(IDX_ROWS, IDX_COLS)
    mesh = plsc.VectorSubcoreMesh(core_axis_name="c", subcore_axis_name="s",
                                  num_cores=NC, num_subcores=NS)
    psum, pcnt = pl.kernel(
        _sc_body,
        out_type=(jax.ShapeDtypeStruct((NC, SEG_PAD, D), jnp.float32),
                  jax.ShapeDtypeStruct((NC, NS, SEG_PAD), jnp.float32)),
        mesh=mesh,
        compiler_params=pltpu.CompilerParams(use_tc_tiling_on_sc=False,
                                             needs_layout_passes=False),
        scratch_types=[
            pltpu.VMEM_SHARED((SEG_PAD, D), jnp.float32),
            pltpu.VMEM((CHUNK, D), jnp.float32),
            pltpu.VMEM((CHUNK, D), jnp.float32),
            pltpu.VMEM((1, IDX_COLS), jnp.int32),
            pltpu.VMEM((1, IDX_COLS), jnp.int32),
            pltpu.VMEM((SEG_PAD,), jnp.float32),
            pltpu.SemaphoreType.DMA,
            pltpu.SemaphoreType.DMA,
            pltpu.SemaphoreType.DMA,
            pltpu.SemaphoreType.DMA,
            pltpu.SemaphoreType.DMA,
            pltpu.SemaphoreType.DMA,
        ],
    )(msg, idx2d)

    cnts = pcnt.reshape(NC * NS, SEG_PAD)
    out = pl.pallas_call(
        _combine_body,
        grid=(SEG_PAD // N_BLK,),
        in_specs=[
            pl.BlockSpec((NC, N_BLK, D), lambda i: (0, i, 0)),
            pl.BlockSpec((NC * NS, N_BLK), lambda i: (0, i)),
        ],
        out_specs=pl.BlockSpec((N_BLK, D), lambda i: (i, 0)),
        out_shape=jax.ShapeDtypeStruct((SEG_PAD, D), jnp.float32),
    )(psum, cnts)
    return out[:NSEG]
